# Initial kernel scaffold; baseline (speedup 1.0000x reference)
import jax
import jax.numpy as jnp
from jax.experimental import pallas as pl


def kernel(x, edge_index, batch, W1, b1, W2, b2, W3, b3, Wl, bl):
    raise NotImplementedError("write your pallas kernel here")


# speedup vs baseline: 7.7795x; 7.7795x over previous
import functools

import jax
import jax.numpy as jnp
from jax import lax
from jax.experimental import pallas as pl
from jax.experimental.pallas import tpu as pltpu
from jax.experimental.pallas import tpu_sc as plsc

NC = 2
NS = 16
L = 16
NG = 256
EBLK = 128


def _cdiv(a, b):
    return (a + b - 1) // b


def _mesh():
    return plsc.VectorSubcoreMesh(core_axis_name="c", subcore_axis_name="s")


def _make_deg(NP, EP):
    Et = EP // (NC * NS)
    NB = Et // EBLK
    R = NP // NS

    @functools.partial(
        pl.kernel,
        out_type=jax.ShapeDtypeStruct((2 * NP,), jnp.float32),
        mesh=_mesh(),
        compiler_params=pltpu.CompilerParams(use_tc_tiling_on_sc=False),
        scratch_types=[
            pltpu.VMEM((EBLK,), jnp.int32),
            pltpu.VMEM((EBLK,), jnp.float32),
            pltpu.VMEM((R,), jnp.float32),
            pltpu.VMEM_SHARED((NP,), jnp.float32),
        ],
    )
    def deg_kernel(dst, out, didx, ones, zbuf, accum):
        c = lax.axis_index("c")
        s = lax.axis_index("s")
        for k in range(EBLK // L):
            ones[pl.ds(k * L, L)] = jnp.ones((L,), jnp.float32)

        def zfill(i, carry):
            zbuf[pl.ds(i * L, L)] = jnp.zeros((L,), jnp.float32)
            return carry

        lax.fori_loop(0, R // L, zfill, 0)
        pltpu.sync_copy(zbuf, accum.at[pl.ds(s * R, R)])
        plsc.subcore_barrier()
        ebase = (s * NC + c) * Et

        def body(j, carry):
            e0 = ebase + j * EBLK
            pltpu.sync_copy(dst.at[pl.ds(e0, EBLK)], didx)
            pltpu.sync_copy(ones, accum.at[didx], add=True)
            return carry

        lax.fori_loop(0, NB, body, 0)
        plsc.subcore_barrier()
        pltpu.sync_copy(accum.at[pl.ds(s * R, R)], zbuf)
        pltpu.sync_copy(zbuf, out.at[pl.ds(c * NP + s * R, R)])

    return deg_kernel


def _make_agg(C, NP, EP):
    split_edges = C == 1
    OUTC = 2 if C == 1 else C
    CPC = 1 if C == 1 else C // NC
    Et = EP // (NC * NS) if split_edges else EP // NS
    NB = Et // EBLK
    R = NP // NS

    ZR = R // 8

    @functools.partial(
        pl.kernel,
        out_type=jax.ShapeDtypeStruct((OUTC * NP, L), jnp.float32),
        mesh=_mesh(),
        compiler_params=pltpu.CompilerParams(use_tc_tiling_on_sc=False),
        scratch_types=[
            pltpu.VMEM((EBLK,), jnp.int32),
            pltpu.VMEM((EBLK,), jnp.int32),
            pltpu.VMEM((EBLK, L), jnp.float32),
            pltpu.VMEM((ZR, L), jnp.float32),
            pltpu.VMEM((ZR, L), jnp.float32),
            pltpu.VMEM_SHARED((NP, L), jnp.float32),
            pltpu.SemaphoreType.DMA,
        ],
    )
    def agg_kernel(table, src, dst, out, sidx, didx, rows, zbuf, stage,
                   accum, gsem):
        c = lax.axis_index("c")
        s = lax.axis_index("s")
        if split_edges:
            ebase = (s * NC + c) * Et
        else:
            ebase = s * Et

        def zfill(i, carry):
            zbuf[i, :] = jnp.zeros((L,), jnp.float32)
            return carry

        lax.fori_loop(0, ZR, zfill, 0)

        for i in range(CPC):
            if C == 1:
                occ = c
                row_off = 0
            else:
                cc = c + i * NC
                occ = cc
                row_off = cc * NP
            for z in range(8):
                pltpu.sync_copy(zbuf, accum.at[pl.ds(s * R + z * ZR, ZR)])
            plsc.subcore_barrier()

            def body(j, carry):
                e0 = ebase + j * EBLK
                pltpu.sync_copy(src.at[pl.ds(e0, EBLK)], sidx)
                pltpu.sync_copy(dst.at[pl.ds(e0, EBLK)], didx)
                if C > 1:
                    for k in range(EBLK // L):
                        sl = pl.ds(k * L, L)
                        sidx[sl] = sidx[sl] + row_off
                pltpu.async_copy(table.at[sidx], rows, gsem).wait()
                pltpu.sync_copy(rows, accum.at[didx], add=True)
                return carry

            lax.fori_loop(0, NB, body, 0)
            plsc.subcore_barrier()
            for z in range(8):
                pltpu.sync_copy(accum.at[pl.ds(s * R + z * ZR, ZR)], stage)
                pltpu.sync_copy(stage,
                                out.at[pl.ds(occ * NP + s * R + z * ZR, ZR)])
            if CPC > 1 and i + 1 < CPC:
                plsc.subcore_barrier()

    return agg_kernel


def _tc1(deg2, xpad, NP, B):
    def body(deg_ref, x_ref, dinv_ref, sq_ref, t0_ref):
        d = deg_ref[0] + deg_ref[1] + 1.0
        di = lax.rsqrt(d)
        dinv_ref[...] = di
        sq_ref[...] = di * di
        t0_ref[...] = x_ref[...] * di

    return pl.pallas_call(
        body,
        grid=(NP // B,),
        in_specs=[
            pl.BlockSpec((2, B, 1), lambda i: (0, i, 0)),
            pl.BlockSpec((B, L), lambda i: (i, 0)),
        ],
        out_specs=[
            pl.BlockSpec((B, 1), lambda i: (i, 0)),
            pl.BlockSpec((B, 1), lambda i: (i, 0)),
            pl.BlockSpec((B, L), lambda i: (i, 0)),
        ],
        out_shape=[
            jax.ShapeDtypeStruct((NP, 1), jnp.float32),
            jax.ShapeDtypeStruct((NP, 1), jnp.float32),
            jax.ShapeDtypeStruct((NP, L), jnp.float32),
        ],
    )(deg2, xpad)


def _tc2(s0, xpad, dinv, sq, W1p, b1, NP, B):
    def body(s0_ref, x_ref, dinv_ref, sq_ref, w_ref, b_ref, t1_ref, u1_ref):
        di = dinv_ref[...]
        p = (s0_ref[0] + s0_ref[1]) * di + x_ref[...] * sq_ref[...]
        h = jnp.dot(p, w_ref[...], preferred_element_type=jnp.float32)
        h = jnp.maximum(h + b_ref[...], 0.0)
        t = h * di
        for cc in range(4):
            t1_ref[cc] = t[:, cc * L:(cc + 1) * L]
        u1_ref[...] = h * sq_ref[...]

    return pl.pallas_call(
        body,
        grid=(NP // B,),
        in_specs=[
            pl.BlockSpec((2, B, L), lambda i: (0, i, 0)),
            pl.BlockSpec((B, L), lambda i: (i, 0)),
            pl.BlockSpec((B, 1), lambda i: (i, 0)),
            pl.BlockSpec((B, 1), lambda i: (i, 0)),
            pl.BlockSpec((L, 64), lambda i: (0, 0)),
            pl.BlockSpec((1, 64), lambda i: (0, 0)),
        ],
        out_specs=[
            pl.BlockSpec((4, B, L), lambda i: (0, i, 0)),
            pl.BlockSpec((B, 64), lambda i: (i, 0)),
        ],
        out_shape=[
            jax.ShapeDtypeStruct((4, NP, L), jnp.float32),
            jax.ShapeDtypeStruct((NP, 64), jnp.float32),
        ],
    )(s0, xpad, dinv, sq, W1p, b1)


def _tc3(s1, u1, dinv, sq, W2, b2, W3, NP, B):
    def body(s1_ref, u1_ref, dinv_ref, sq_ref, w2_ref, b2_ref, w3_ref,
             t2_ref, u2_ref):
        di = dinv_ref[...]
        agg = jnp.concatenate([s1_ref[0], s1_ref[1], s1_ref[2], s1_ref[3]],
                              axis=1)
        a = agg * di + u1_ref[...]
        h2 = jnp.dot(a, w2_ref[...], preferred_element_type=jnp.float32)
        h2 = jnp.maximum(h2 + b2_ref[...], 0.0)
        g = jnp.dot(h2, w3_ref[...], preferred_element_type=jnp.float32)
        t = g * di
        t2_ref[0] = t[:, :L]
        t2_ref[1] = t[:, L:]
        u2_ref[...] = g * sq_ref[...]

    return pl.pallas_call(
        body,
        grid=(NP // B,),
        in_specs=[
            pl.BlockSpec((4, B, L), lambda i: (0, i, 0)),
            pl.BlockSpec((B, 64), lambda i: (i, 0)),
            pl.BlockSpec((B, 1), lambda i: (i, 0)),
            pl.BlockSpec((B, 1), lambda i: (i, 0)),
            pl.BlockSpec((64, 64), lambda i: (0, 0)),
            pl.BlockSpec((1, 64), lambda i: (0, 0)),
            pl.BlockSpec((64, 32), lambda i: (0, 0)),
        ],
        out_specs=[
            pl.BlockSpec((2, B, L), lambda i: (0, i, 0)),
            pl.BlockSpec((B, 32), lambda i: (i, 0)),
        ],
        out_shape=[
            jax.ShapeDtypeStruct((2, NP, L), jnp.float32),
            jax.ShapeDtypeStruct((NP, 32), jnp.float32),
        ],
    )(s1, u1, dinv, sq, W2, b2, W3)


def _tc4(s2, u2, dinv, b3, batchp, Wl, bl, NP, B):
    def body(s2_ref, u2_ref, dinv_ref, b3_ref, batch_ref, wl_ref, bl_ref,
             out_ref, psum, pcnt):
        i = pl.program_id(0)

        @pl.when(i == 0)
        def _init():
            psum[...] = jnp.zeros_like(psum)
            pcnt[...] = jnp.zeros_like(pcnt)

        agg = jnp.concatenate([s2_ref[0], s2_ref[1]], axis=1)
        h3 = agg * dinv_ref[...] + u2_ref[...] + b3_ref[...]
        h3 = jnp.maximum(h3, 0.0)
        onehot = (batch_ref[...] == lax.broadcasted_iota(
            jnp.int32, (1, NG), 1)).astype(jnp.float32)
        psum[...] += lax.dot_general(
            onehot, h3, (((0,), (0,)), ((), ())),
            preferred_element_type=jnp.float32)
        pcnt[...] += jnp.sum(onehot, axis=0, keepdims=True)
        pooled = psum[...] / jnp.maximum(pcnt[...], 1.0).reshape(NG, 1)
        out_ref[...] = jnp.dot(pooled, wl_ref[...],
                               preferred_element_type=jnp.float32) + bl_ref[...]

    return pl.pallas_call(
        body,
        grid=(NP // B,),
        in_specs=[
            pl.BlockSpec((2, B, L), lambda i: (0, i, 0)),
            pl.BlockSpec((B, 32), lambda i: (i, 0)),
            pl.BlockSpec((B, 1), lambda i: (i, 0)),
            pl.BlockSpec((1, 32), lambda i: (0, 0)),
            pl.BlockSpec((B, 1), lambda i: (i, 0)),
            pl.BlockSpec((32, 2), lambda i: (0, 0)),
            pl.BlockSpec((1, 2), lambda i: (0, 0)),
        ],
        out_specs=pl.BlockSpec((NG, 2), lambda i: (0, 0)),
        out_shape=jax.ShapeDtypeStruct((NG, 2), jnp.float32),
        scratch_shapes=[
            pltpu.VMEM((NG, 32), jnp.float32),
            pltpu.VMEM((1, NG), jnp.float32),
        ],
    )(s2, u2, dinv, b3, batchp, Wl, bl)


def kernel(x, edge_index, batch, W1, b1, W2, b2, W3, b3, Wl, bl):
    N = x.shape[0]
    F = x.shape[1]
    E = edge_index.shape[1]
    NP = _cdiv(N + 1, 1024) * 1024
    EP = _cdiv(E, NC * NS * EBLK) * (NC * NS * EBLK)
    B = 1024

    pad_e = EP - E
    srcp = jnp.concatenate(
        [edge_index[0].astype(jnp.int32),
         jnp.full((pad_e,), N, jnp.int32)])
    dstp = jnp.concatenate(
        [edge_index[1].astype(jnp.int32),
         jnp.full((pad_e,), N, jnp.int32)])
    xpad = jnp.pad(x, ((0, NP - N), (0, L - F)))
    W1p = jnp.pad(W1, ((0, L - F), (0, 0)))
    batchp = jnp.pad(batch.astype(jnp.int32), (0, NP - N),
                     constant_values=NG).reshape(NP, 1)

    deg2 = _make_deg(NP, EP)(dstp).reshape(2, NP, 1)
    dinv, sq, t0 = _tc1(deg2, xpad, NP, B)

    s0 = _make_agg(1, NP, EP)(t0, srcp, dstp).reshape(2, NP, L)
    t1, u1 = _tc2(s0, xpad, dinv, sq, W1p, b1.reshape(1, 64), NP, B)

    s1 = _make_agg(4, NP, EP)(t1.reshape(4 * NP, L), srcp,
                              dstp).reshape(4, NP, L)
    t2, u2 = _tc3(s1, u1, dinv, sq, W2, b2.reshape(1, 64), W3, NP, B)

    s2 = _make_agg(2, NP, EP)(t2.reshape(2 * NP, L), srcp,
                              dstp).reshape(2, NP, L)

    return _tc4(s2, u2, dinv, b3.reshape(1, 32), batchp, Wl,
                bl.reshape(1, 2), NP, B)
